# baseline (speedup 1.0000x reference)
import functools

import jax
import jax.numpy as jnp
from jax import lax
from jax.experimental import pallas as pl
from jax.experimental.pallas import tpu as pltpu
from jax.experimental.pallas import tpu_sc as plsc

N_NODES = 10000
D = 256
DH = 128
N_TILES = 16
NB = 162
KB = 64
E_PAD = N_TILES * NB * KB
N_ACC = 10240
ROWS_PER_TILE = N_ACC // N_TILES
ZCHUNKS = ROWS_PER_TILE // KB
NRING = 3
NIDX = 6
NT = NB // NIDX

_mesh = plsc.VectorSubcoreMesh(core_axis_name="c", subcore_axis_name="s")


@functools.partial(
    pl.kernel,
    out_type=jax.ShapeDtypeStruct((2 * N_ACC, DH), jnp.float32),
    mesh=_mesh,
    scratch_types=[
        pltpu.VMEM((NIDX, KB), jnp.int32),
        pltpu.VMEM((NIDX, KB), jnp.int32),
        pltpu.VMEM((NIDX, KB), jnp.float32),
        pltpu.VMEM((KB, DH), jnp.float32),
        pltpu.VMEM((KB, DH), jnp.float32),
        pltpu.VMEM((KB, DH), jnp.float32),
        pltpu.VMEM_SHARED((N_ACC, DH), jnp.float32),
        [pltpu.SemaphoreType.DMA] * NRING,
        [pltpu.SemaphoreType.DMA] * NRING,
        [pltpu.SemaphoreType.DMA] * NIDX,
    ],
)
def _sc_segment(xs_hbm, isrc_hbm, idst_hbm, w_hbm, out_hbm,
                isrc_v, idst_v, w_v, rows0, rows1, rows2, acc,
                gsems, ssems, isems):
    c = lax.axis_index("c")
    s = lax.axis_index("s")
    rows_bufs = (rows0, rows1, rows2)

    off = jnp.broadcast_to((c * N_NODES).astype(jnp.int32), (16,))

    zero = jnp.zeros((16,), jnp.float32)

    @pl.loop(0, KB)
    def _zr(e):
        for r in range(DH // 16):
            rows0[e, pl.ds(16 * r, 16)] = zero

    for j in range(ZCHUNKS):
        pltpu.sync_copy(rows0, acc.at[pl.ds((s * ZCHUNKS + j) * KB, KB)])
    plsc.subcore_barrier()

    def start_idx_load(b, slot):
        pltpu.async_copy(isrc_hbm.at[s, b], isrc_v.at[slot], isems[slot])
        pltpu.async_copy(idst_hbm.at[s, b], idst_v.at[slot], isems[slot])
        pltpu.async_copy(w_hbm.at[s, b], w_v.at[slot], isems[slot])

    def wait_idx_load(b, slot):
        pltpu.make_async_copy(
            isrc_hbm.at[s, b], isrc_v.at[slot], isems[slot]).wait()
        pltpu.make_async_copy(
            idst_hbm.at[s, b], idst_v.at[slot], isems[slot]).wait()
        pltpu.make_async_copy(
            w_hbm.at[s, b], w_v.at[slot], isems[slot]).wait()
        for g in range(KB // 16):
            sl = pl.ds(16 * g, 16)
            isrc_v[slot, sl] = isrc_v[slot, sl] + off

    def start_gather(slot, rbuf, rsem):
        pltpu.async_copy(xs_hbm.at[isrc_v.at[slot]], rbuf, rsem)

    def scale(slot, rows):
        @pl.loop(0, KB // 16)
        def _scale(g):
            wvec = w_v[slot, pl.ds(16 * g, 16)]
            for j in range(16):
                wj = jnp.broadcast_to(wvec[j], (16,))
                e = 16 * g + j
                for r in range(DH // 16):
                    sl = pl.ds(16 * r, 16)
                    rows[e, sl] = rows[e, sl] * wj

    for b in range(4):
        start_idx_load(b, b)
    for b in range(2):
        wait_idx_load(b, b)
        start_gather(b, rows_bufs[b], gsems[b])

    @pl.loop(0, NT)
    def _t(t):
        for j6 in range(NIDX):
            b = NIDX * t + j6
            j = j6 % NRING
            jn = (j6 + 2) % NRING
            rows = rows_bufs[j]
            pltpu.make_async_copy(
                xs_hbm.at[isrc_v.at[j6]], rows, gsems[j]).wait()
            scale(j6, rows)
            pltpu.async_copy(rows, acc.at[idst_v.at[j6]], ssems[j], add=True)

            @pl.when(b >= 1)
            def _():
                pltpu.make_async_copy(
                    rows_bufs[jn], acc.at[idst_v.at[(j6 + 5) % NIDX]],
                    ssems[jn]).wait()

            @pl.when(b + 2 < NB)
            def _():
                wait_idx_load(b + 2, (j6 + 2) % NIDX)
                start_gather((j6 + 2) % NIDX, rows_bufs[jn], gsems[jn])

            @pl.when(b + 4 < NB)
            def _():
                start_idx_load(b + 4, (j6 + 4) % NIDX)

    pltpu.make_async_copy(
        rows_bufs[(NB - 1) % NRING], acc.at[idst_v.at[(NB - 1) % NIDX]],
        ssems[(NB - 1) % NRING]).wait()
    plsc.subcore_barrier()

    base = c * N_ACC + s * ROWS_PER_TILE
    pltpu.sync_copy(acc.at[pl.ds(s * ROWS_PER_TILE, ROWS_PER_TILE)],
                    out_hbm.at[pl.ds(base, ROWS_PER_TILE)])


def _tc_body(xd_ref, g_ref, wlt_ref, wit_ref, out_ref):
    xd = xd_ref[...]
    g = g_ref[...]
    y = jnp.dot(xd + g, wlt_ref[...], preferred_element_type=jnp.float32)
    y += jnp.dot(xd * g, wit_ref[...], preferred_element_type=jnp.float32)
    out_ref[...] = jnp.where(y >= 0, y, 0.01 * y)


_TR = 512


def _tc_post(xd_pad, g_pad, wlt, wit):
    return pl.pallas_call(
        _tc_body,
        grid=(N_ACC // _TR,),
        in_specs=[
            pl.BlockSpec((_TR, D), lambda i: (i, 0)),
            pl.BlockSpec((_TR, D), lambda i: (i, 0)),
            pl.BlockSpec((D, D), lambda i: (0, 0)),
            pl.BlockSpec((D, D), lambda i: (0, 0)),
        ],
        out_specs=pl.BlockSpec((_TR, D), lambda i: (i, 0)),
        out_shape=jax.ShapeDtypeStruct((N_ACC, D), jnp.float32),
    )(xd_pad, g_pad, wlt, wit)


@jax.jit
def kernel(x_src, x_dst, edge_index, edge_weight, W_loop, W_intr):
    E = edge_index.shape[1]
    i_src = edge_index[0].astype(jnp.int32)
    i_dst = edge_index[1].astype(jnp.int32)
    w = edge_weight[:, 0]

    pad = E_PAD - E
    i_src_p = jnp.pad(i_src, (0, pad)).reshape(N_TILES, NB, KB)
    i_dst_p = jnp.pad(i_dst, (0, pad)).reshape(N_TILES, NB, KB)
    w_p = jnp.pad(w, (0, pad)).reshape(N_TILES, NB, KB)

    xs = jnp.concatenate([x_src[:, :DH], x_src[:, DH:]], axis=0)

    out = _sc_segment(xs, i_src_p, i_dst_p, w_p)
    g = jnp.concatenate(
        [out[:N_NODES], out[N_ACC:N_ACC + N_NODES]], axis=1)

    g_pad = jnp.pad(g, ((0, N_ACC - N_NODES), (0, 0)))
    xd_pad = jnp.pad(x_dst, ((0, N_ACC - N_NODES), (0, 0)))
    res = _tc_post(xd_pad, g_pad, W_loop.T, W_intr.T)
    return res[:N_NODES]


# speedup vs baseline: 1.1268x; 1.1268x over previous
import functools

import jax
import jax.numpy as jnp
from jax import lax
from jax.experimental import pallas as pl
from jax.experimental.pallas import tpu as pltpu
from jax.experimental.pallas import tpu_sc as plsc

N_NODES = 10000
D = 256
DH = 128
N_TILES = 16
NB = 80
KB = 128
E_PAD = N_TILES * NB * KB
SB = 8
NSB = NB // SB
N_ACC = 10240
ROWS_PER_TILE = N_ACC // N_TILES
ZCHUNKS = ROWS_PER_TILE // KB

_mesh = plsc.VectorSubcoreMesh(core_axis_name="c", subcore_axis_name="s")


@functools.partial(
    pl.kernel,
    out_type=jax.ShapeDtypeStruct((2 * N_ACC, DH), jnp.float32),
    mesh=_mesh,
    scratch_types=[
        pltpu.VMEM((2, SB, KB), jnp.int32),
        pltpu.VMEM((2, SB, KB), jnp.int32),
        pltpu.VMEM((2, SB, KB), jnp.float32),
        pltpu.VMEM((KB, DH), jnp.float32),
        pltpu.VMEM((KB, DH), jnp.float32),
        pltpu.VMEM_SHARED((N_ACC, DH), jnp.float32),
        [pltpu.SemaphoreType.DMA] * 2,
        [pltpu.SemaphoreType.DMA] * 2,
        [pltpu.SemaphoreType.DMA] * 2,
    ],
)
def _sc_segment(xs_hbm, isrc_hbm, idst_hbm, w_hbm, out_hbm,
                isrc_v, idst_v, w_v, rows0, rows1, acc,
                gsems, ssems, isems):
    c = lax.axis_index("c")
    s = lax.axis_index("s")
    rbufs = (rows0, rows1)

    off = jnp.broadcast_to((c * N_NODES).astype(jnp.int32), (16,))
    zero = jnp.zeros((16,), jnp.float32)

    @pl.loop(0, KB)
    def _zr(e):
        for r in range(DH // 16):
            rows0[e, pl.ds(16 * r, 16)] = zero

    for j in range(ZCHUNKS):
        pltpu.sync_copy(rows0, acc.at[pl.ds((s * ZCHUNKS + j) * KB, KB)])
    plsc.subcore_barrier()

    def start_sb(g, slot):
        src = pl.ds(g * SB, SB)
        pltpu.async_copy(isrc_hbm.at[s, src], isrc_v.at[slot], isems[slot])
        pltpu.async_copy(idst_hbm.at[s, src], idst_v.at[slot], isems[slot])
        pltpu.async_copy(w_hbm.at[s, src], w_v.at[slot], isems[slot])

    def wait_sb(g, slot):
        src = pl.ds(g * SB, SB)
        pltpu.make_async_copy(
            isrc_hbm.at[s, src], isrc_v.at[slot], isems[slot]).wait()
        pltpu.make_async_copy(
            idst_hbm.at[s, src], idst_v.at[slot], isems[slot]).wait()
        pltpu.make_async_copy(
            w_hbm.at[s, src], w_v.at[slot], isems[slot]).wait()
        @pl.loop(0, SB)
        def _adj(bb):
            for q in range(KB // 16):
                sl = pl.ds(16 * q, 16)
                isrc_v[slot, bb, sl] = isrc_v[slot, bb, sl] + off

    def start_gather(slot, row, j):
        pltpu.async_copy(xs_hbm.at[isrc_v.at[slot, row]], rbufs[j], gsems[j])

    def wait_gather(slot, row, j):
        pltpu.make_async_copy(
            xs_hbm.at[isrc_v.at[slot, row]], rbufs[j], gsems[j]).wait()

    def start_scatter(slot, row, j):
        pltpu.async_copy(
            rbufs[j], acc.at[idst_v.at[slot, row]], ssems[j], add=True)

    def wait_scatter(slot, row, j):
        pltpu.make_async_copy(
            rbufs[j], acc.at[idst_v.at[slot, row]], ssems[j]).wait()

    def scale(slot, k, j):
        rows = rbufs[j]

        @pl.loop(0, KB // 16)
        def _sc(q):
            wvec = w_v[slot, k, pl.ds(16 * q, 16)]
            for i in range(16):
                wj = jnp.broadcast_to(wvec[i], (16,))
                e = 16 * q + i
                for u in range(DH // 16):
                    sl = pl.ds(16 * u, 16)
                    rows[e, sl] = rows[e, sl] * wj

    start_sb(0, 0)
    wait_sb(0, 0)
    start_sb(1, 1)
    start_gather(0, 0, 0)

    @pl.loop(0, NSB // 2)
    def _gg(gg):
        for g2 in range(2):
            slot = g2
            nslot = (g2 + 1) % 2
            for k in range(SB):
                j = k % 2
                nj = (k + 1) % 2
                wait_gather(slot, k, j)
                scale(slot, k, j)
                start_scatter(slot, k, j)
                if (g2, k) == (0, 0):
                    @pl.when(gg >= 1)
                    def _():
                        wait_scatter(1, SB - 1, nj)
                elif k == 0:
                    wait_scatter(nslot, SB - 1, nj)
                else:
                    wait_scatter(slot, k - 1, nj)
                if k == 1:
                    if g2 == 0:
                        @pl.when(gg >= 1)
                        def _():
                            start_sb(2 * gg + 1, nslot)
                    else:
                        @pl.when(gg < NSB // 2 - 1)
                        def _():
                            start_sb(2 * gg + 2, nslot)
                if k == SB - 2:
                    if g2 == 0:
                        wait_sb(2 * gg + 1, nslot)
                    else:
                        @pl.when(gg < NSB // 2 - 1)
                        def _():
                            wait_sb(2 * gg + 2, nslot)
                if k < SB - 1:
                    start_gather(slot, k + 1, nj)
                elif g2 == 0:
                    start_gather(nslot, 0, nj)
                else:
                    @pl.when(gg < NSB // 2 - 1)
                    def _():
                        start_gather(nslot, 0, nj)

    wait_scatter(1, SB - 1, (NB - 1) % 2)
    plsc.subcore_barrier()

    base = c * N_ACC + s * ROWS_PER_TILE
    pltpu.sync_copy(acc.at[pl.ds(s * ROWS_PER_TILE, ROWS_PER_TILE)],
                    out_hbm.at[pl.ds(base, ROWS_PER_TILE)])


def _tc_body(xd_ref, g_ref, wlt_ref, wit_ref, out_ref):
    xd = xd_ref[...]
    g = g_ref[...]
    y = jnp.dot(xd + g, wlt_ref[...], preferred_element_type=jnp.float32)
    y += jnp.dot(xd * g, wit_ref[...], preferred_element_type=jnp.float32)
    out_ref[...] = jnp.where(y >= 0, y, 0.01 * y)


_TR = 512


def _tc_post(xd_pad, g_pad, wlt, wit):
    return pl.pallas_call(
        _tc_body,
        grid=(N_ACC // _TR,),
        in_specs=[
            pl.BlockSpec((_TR, D), lambda i: (i, 0)),
            pl.BlockSpec((_TR, D), lambda i: (i, 0)),
            pl.BlockSpec((D, D), lambda i: (0, 0)),
            pl.BlockSpec((D, D), lambda i: (0, 0)),
        ],
        out_specs=pl.BlockSpec((_TR, D), lambda i: (i, 0)),
        out_shape=jax.ShapeDtypeStruct((N_ACC, D), jnp.float32),
    )(xd_pad, g_pad, wlt, wit)


@jax.jit
def kernel(x_src, x_dst, edge_index, edge_weight, W_loop, W_intr):
    E = edge_index.shape[1]
    i_src = edge_index[0].astype(jnp.int32)
    i_dst = edge_index[1].astype(jnp.int32)
    w = edge_weight[:, 0]

    pad = E_PAD - E
    i_src_p = jnp.pad(i_src, (0, pad)).reshape(N_TILES, NB, KB)
    i_dst_p = jnp.pad(i_dst, (0, pad)).reshape(N_TILES, NB, KB)
    w_p = jnp.pad(w, (0, pad)).reshape(N_TILES, NB, KB)

    xs = jnp.concatenate([x_src[:, :DH], x_src[:, DH:]], axis=0)

    out = _sc_segment(xs, i_src_p, i_dst_p, w_p)
    g = jnp.concatenate(
        [out[:N_NODES], out[N_ACC:N_ACC + N_NODES]], axis=1)

    g_pad = jnp.pad(g, ((0, N_ACC - N_NODES), (0, 0)))
    xd_pad = jnp.pad(x_dst, ((0, N_ACC - N_NODES), (0, 0)))
    res = _tc_post(xd_pad, g_pad, W_loop.T, W_intr.T)
    return res[:N_NODES]


# speedup vs baseline: 1.2527x; 1.1118x over previous
import functools

import jax
import jax.numpy as jnp
from jax import lax
from jax.experimental import pallas as pl
from jax.experimental.pallas import tpu as pltpu
from jax.experimental.pallas import tpu_sc as plsc

N_NODES = 10000
D = 256
DH = 128
N_TILES = 16
NB = 80
KB = 128
E_PAD = N_TILES * NB * KB
SB = 8
NSB = NB // SB
N_ACC = 10240
ROWS_PER_TILE = N_ACC // N_TILES
ZCHUNKS = ROWS_PER_TILE // KB

_mesh = plsc.VectorSubcoreMesh(core_axis_name="c", subcore_axis_name="s")


@functools.partial(
    pl.kernel,
    out_type=jax.ShapeDtypeStruct((2 * N_ACC, DH), jnp.float32),
    mesh=_mesh,
    scratch_types=[
        pltpu.VMEM((2, SB, KB), jnp.int32),
        pltpu.VMEM((2, SB, KB), jnp.int32),
        pltpu.VMEM((2, SB, KB), jnp.float32),
        pltpu.VMEM((KB, DH), jnp.float32),
        pltpu.VMEM((KB, DH), jnp.float32),
        pltpu.VMEM_SHARED((N_ACC, DH), jnp.float32),
        [pltpu.SemaphoreType.DMA] * 2,
        [pltpu.SemaphoreType.DMA] * 2,
    ],
)
def _sc_segment(xs_hbm, isrc_hbm, idst_hbm, w_hbm, out_hbm,
                isrc_v, idst_v, w_v, rows0, rows1, acc,
                gsems, isems):
    c = lax.axis_index("c")
    s = lax.axis_index("s")
    rbufs = (rows0, rows1)

    off = jnp.broadcast_to((c * N_NODES).astype(jnp.int32), (16,))
    zero = jnp.zeros((16,), jnp.float32)

    @pl.loop(0, KB)
    def _zr(e):
        for r in range(DH // 16):
            rows0[e, pl.ds(16 * r, 16)] = zero

    for j in range(ZCHUNKS):
        pltpu.sync_copy(rows0, acc.at[pl.ds((s * ZCHUNKS + j) * KB, KB)])
    plsc.subcore_barrier()

    def start_sb(g, slot):
        src = pl.ds(g * SB, SB)
        pltpu.async_copy(isrc_hbm.at[s, src], isrc_v.at[slot], isems[slot])
        pltpu.async_copy(idst_hbm.at[s, src], idst_v.at[slot], isems[slot])
        pltpu.async_copy(w_hbm.at[s, src], w_v.at[slot], isems[slot])

    def wait_sb(g, slot):
        src = pl.ds(g * SB, SB)
        pltpu.make_async_copy(
            isrc_hbm.at[s, src], isrc_v.at[slot], isems[slot]).wait()
        pltpu.make_async_copy(
            idst_hbm.at[s, src], idst_v.at[slot], isems[slot]).wait()
        pltpu.make_async_copy(
            w_hbm.at[s, src], w_v.at[slot], isems[slot]).wait()
        @pl.loop(0, SB)
        def _adj(bb):
            for q in range(KB // 16):
                sl = pl.ds(16 * q, 16)
                isrc_v[slot, bb, sl] = isrc_v[slot, bb, sl] + off

    def start_gather(slot, row, j):
        pltpu.async_copy(xs_hbm.at[isrc_v.at[slot, row]], rbufs[j], gsems[j])

    def wait_gather(slot, row, j):
        pltpu.make_async_copy(
            xs_hbm.at[isrc_v.at[slot, row]], rbufs[j], gsems[j]).wait()

    def sync_scatter(slot, row, j):
        pltpu.sync_copy(rbufs[j], acc.at[idst_v.at[slot, row]], add=True)

    def scale(slot, k, j):
        rows = rbufs[j]

        @pl.loop(0, KB // 16)
        def _sc(q):
            wvec = w_v[slot, k, pl.ds(16 * q, 16)]
            for i in range(16):
                wj = jnp.broadcast_to(wvec[i], (16,))
                e = 16 * q + i
                for u in range(DH // 16):
                    sl = pl.ds(16 * u, 16)
                    rows[e, sl] = rows[e, sl] * wj

    start_sb(0, 0)
    wait_sb(0, 0)
    start_sb(1, 1)
    start_gather(0, 0, 0)

    @pl.loop(0, NSB // 2)
    def _gg(gg):
        for g2 in range(2):
            slot = g2
            nslot = (g2 + 1) % 2
            for k in range(SB):
                j = k % 2
                nj = (k + 1) % 2
                wait_gather(slot, k, j)
                if k == SB - 2:
                    start_gather(slot, k + 1, nj)
                    if g2 == 0:
                        wait_sb(2 * gg + 1, nslot)
                    else:
                        @pl.when(gg < NSB // 2 - 1)
                        def _():
                            wait_sb(2 * gg + 2, nslot)
                elif k < SB - 1:
                    start_gather(slot, k + 1, nj)
                elif g2 == 0:
                    start_gather(nslot, 0, nj)
                else:
                    @pl.when(gg < NSB // 2 - 1)
                    def _():
                        start_gather(nslot, 0, nj)
                scale(slot, k, j)
                sync_scatter(slot, k, j)
                if k == 1:
                    if g2 == 0:
                        @pl.when(gg >= 1)
                        def _():
                            start_sb(2 * gg + 1, nslot)
                    else:
                        @pl.when(gg < NSB // 2 - 1)
                        def _():
                            start_sb(2 * gg + 2, nslot)

    plsc.subcore_barrier()

    base = c * N_ACC + s * ROWS_PER_TILE
    pltpu.sync_copy(acc.at[pl.ds(s * ROWS_PER_TILE, ROWS_PER_TILE)],
                    out_hbm.at[pl.ds(base, ROWS_PER_TILE)])


def _tc_body(xd_ref, g_ref, wlt_ref, wit_ref, out_ref):
    xd = xd_ref[...]
    g = g_ref[...]
    y = jnp.dot(xd + g, wlt_ref[...], preferred_element_type=jnp.float32)
    y += jnp.dot(xd * g, wit_ref[...], preferred_element_type=jnp.float32)
    out_ref[...] = jnp.where(y >= 0, y, 0.01 * y)


_TR = 512


def _tc_post(xd_pad, g_pad, wlt, wit):
    return pl.pallas_call(
        _tc_body,
        grid=(N_ACC // _TR,),
        in_specs=[
            pl.BlockSpec((_TR, D), lambda i: (i, 0)),
            pl.BlockSpec((_TR, D), lambda i: (i, 0)),
            pl.BlockSpec((D, D), lambda i: (0, 0)),
            pl.BlockSpec((D, D), lambda i: (0, 0)),
        ],
        out_specs=pl.BlockSpec((_TR, D), lambda i: (i, 0)),
        out_shape=jax.ShapeDtypeStruct((N_ACC, D), jnp.float32),
    )(xd_pad, g_pad, wlt, wit)


@jax.jit
def kernel(x_src, x_dst, edge_index, edge_weight, W_loop, W_intr):
    E = edge_index.shape[1]
    i_src = edge_index[0].astype(jnp.int32)
    i_dst = edge_index[1].astype(jnp.int32)
    w = edge_weight[:, 0]

    pad = E_PAD - E
    i_src_p = jnp.pad(i_src, (0, pad)).reshape(N_TILES, NB, KB)
    i_dst_p = jnp.pad(i_dst, (0, pad)).reshape(N_TILES, NB, KB)
    w_p = jnp.pad(w, (0, pad)).reshape(N_TILES, NB, KB)

    xs = jnp.concatenate([x_src[:, :DH], x_src[:, DH:]], axis=0)

    out = _sc_segment(xs, i_src_p, i_dst_p, w_p)
    g = jnp.concatenate(
        [out[:N_NODES], out[N_ACC:N_ACC + N_NODES]], axis=1)

    g_pad = jnp.pad(g, ((0, N_ACC - N_NODES), (0, 0)))
    xd_pad = jnp.pad(x_dst, ((0, N_ACC - N_NODES), (0, 0)))
    res = _tc_post(xd_pad, g_pad, W_loop.T, W_intr.T)
    return res[:N_NODES]


# speedup vs baseline: 1.2646x; 1.0094x over previous
import functools

import jax
import jax.numpy as jnp
from jax import lax
from jax.experimental import pallas as pl
from jax.experimental.pallas import tpu as pltpu
from jax.experimental.pallas import tpu_sc as plsc

N_NODES = 10000
D = 256
DH = 128
N_TILES = 16
NB = 80
KB = 128
E_PAD = N_TILES * NB * KB
SB = 8
NSB = NB // SB
N_ACC = 10240
ROWS_PER_TILE = N_ACC // N_TILES
ZCHUNKS = ROWS_PER_TILE // KB

_mesh = plsc.VectorSubcoreMesh(core_axis_name="c", subcore_axis_name="s")


@functools.partial(
    pl.kernel,
    out_type=jax.ShapeDtypeStruct((2 * N_ACC, DH), jnp.float32),
    mesh=_mesh,
    scratch_types=[
        pltpu.VMEM((2, SB, KB), jnp.int32),
        pltpu.VMEM((2, SB, KB), jnp.int32),
        pltpu.VMEM((2, SB, KB), jnp.float32),
        pltpu.VMEM((KB, DH), jnp.float32),
        pltpu.VMEM((KB, DH), jnp.float32),
        pltpu.VMEM_SHARED((N_ACC, DH), jnp.float32),
        [pltpu.SemaphoreType.DMA] * 2,
        [pltpu.SemaphoreType.DMA] * 2,
    ],
)
def _sc_segment(xs_hbm, isrc_hbm, idst_hbm, w_hbm, out_hbm,
                isrc_v, idst_v, w_v, rows0, rows1, acc,
                gsems, isems):
    c = lax.axis_index("c")
    s = lax.axis_index("s")
    rbufs = (rows0, rows1)

    off = jnp.broadcast_to(c.astype(jnp.int32), (16,))
    zero = jnp.zeros((16,), jnp.float32)

    @pl.loop(0, KB)
    def _zr(e):
        for r in range(DH // 16):
            rows0[e, pl.ds(16 * r, 16)] = zero

    for j in range(ZCHUNKS):
        pltpu.sync_copy(rows0, acc.at[pl.ds((s * ZCHUNKS + j) * KB, KB)])
    plsc.subcore_barrier()

    def start_sb(g, slot):
        src = pl.ds(g * SB, SB)
        pltpu.async_copy(isrc_hbm.at[s, src], isrc_v.at[slot], isems[slot])
        pltpu.async_copy(idst_hbm.at[s, src], idst_v.at[slot], isems[slot])
        pltpu.async_copy(w_hbm.at[s, src], w_v.at[slot], isems[slot])

    def wait_sb(g, slot):
        src = pl.ds(g * SB, SB)
        pltpu.make_async_copy(
            isrc_hbm.at[s, src], isrc_v.at[slot], isems[slot]).wait()
        pltpu.make_async_copy(
            idst_hbm.at[s, src], idst_v.at[slot], isems[slot]).wait()
        pltpu.make_async_copy(
            w_hbm.at[s, src], w_v.at[slot], isems[slot]).wait()
        @pl.loop(0, SB)
        def _adj(bb):
            for q in range(KB // 16):
                sl = pl.ds(16 * q, 16)
                isrc_v[slot, bb, sl] = isrc_v[slot, bb, sl] * 2 + off

    def start_gather(slot, row, j):
        pltpu.async_copy(xs_hbm.at[isrc_v.at[slot, row]], rbufs[j], gsems[j])

    def wait_gather(slot, row, j):
        pltpu.make_async_copy(
            xs_hbm.at[isrc_v.at[slot, row]], rbufs[j], gsems[j]).wait()

    def sync_scatter(slot, row, j):
        pltpu.sync_copy(rbufs[j], acc.at[idst_v.at[slot, row]], add=True)

    def scale(slot, k, j):
        rows = rbufs[j]

        @pl.loop(0, KB // 16)
        def _sc(q):
            wvec = w_v[slot, k, pl.ds(16 * q, 16)]
            for i in range(16):
                wj = jnp.broadcast_to(wvec[i], (16,))
                e = 16 * q + i
                for u in range(DH // 16):
                    sl = pl.ds(16 * u, 16)
                    rows[e, sl] = rows[e, sl] * wj

    start_sb(0, 0)
    wait_sb(0, 0)
    start_sb(1, 1)
    start_gather(0, 0, 0)

    @pl.loop(0, NSB // 2)
    def _gg(gg):
        for g2 in range(2):
            slot = g2
            nslot = (g2 + 1) % 2
            for k in range(SB):
                j = k % 2
                nj = (k + 1) % 2
                wait_gather(slot, k, j)
                if k == SB - 2:
                    start_gather(slot, k + 1, nj)
                    if g2 == 0:
                        wait_sb(2 * gg + 1, nslot)
                    else:
                        @pl.when(gg < NSB // 2 - 1)
                        def _():
                            wait_sb(2 * gg + 2, nslot)
                elif k < SB - 1:
                    start_gather(slot, k + 1, nj)
                elif g2 == 0:
                    start_gather(nslot, 0, nj)
                else:
                    @pl.when(gg < NSB // 2 - 1)
                    def _():
                        start_gather(nslot, 0, nj)
                scale(slot, k, j)
                sync_scatter(slot, k, j)
                if k == 1:
                    if g2 == 0:
                        @pl.when(gg >= 1)
                        def _():
                            start_sb(2 * gg + 1, nslot)
                    else:
                        @pl.when(gg < NSB // 2 - 1)
                        def _():
                            start_sb(2 * gg + 2, nslot)

    plsc.subcore_barrier()

    base = c * N_ACC + s * ROWS_PER_TILE
    pltpu.sync_copy(acc.at[pl.ds(s * ROWS_PER_TILE, ROWS_PER_TILE)],
                    out_hbm.at[pl.ds(base, ROWS_PER_TILE)])


def _tc_body(xd_ref, glo_ref, ghi_ref, wlt_ref, wit_ref, out_ref):
    xd = xd_ref[...]
    g = jnp.concatenate([glo_ref[...], ghi_ref[...]], axis=1)
    y = jnp.dot(xd + g, wlt_ref[...], preferred_element_type=jnp.float32)
    y += jnp.dot(xd * g, wit_ref[...], preferred_element_type=jnp.float32)
    out_ref[...] = jnp.where(y >= 0, y, 0.01 * y)


_TR = 512


def _tc_post(xd, sc_out, wlt, wit):
    return pl.pallas_call(
        _tc_body,
        grid=(N_ACC // _TR,),
        in_specs=[
            pl.BlockSpec((_TR, D), lambda i: (i, 0)),
            pl.BlockSpec((_TR, DH), lambda i: (i, 0)),
            pl.BlockSpec((_TR, DH), lambda i: (i + N_ACC // _TR, 0)),
            pl.BlockSpec((D, D), lambda i: (0, 0)),
            pl.BlockSpec((D, D), lambda i: (0, 0)),
        ],
        out_specs=pl.BlockSpec((_TR, D), lambda i: (i, 0)),
        out_shape=jax.ShapeDtypeStruct((N_NODES, D), jnp.float32),
    )(xd, sc_out, sc_out, wlt, wit)


@jax.jit
def kernel(x_src, x_dst, edge_index, edge_weight, W_loop, W_intr):
    E = edge_index.shape[1]
    i_src = edge_index[0].astype(jnp.int32)
    i_dst = edge_index[1].astype(jnp.int32)
    w = edge_weight[:, 0]

    pad = E_PAD - E
    i_src_p = jnp.pad(i_src, (0, pad)).reshape(N_TILES, NB, KB)
    i_dst_p = jnp.pad(i_dst, (0, pad)).reshape(N_TILES, NB, KB)
    w_p = jnp.pad(w, (0, pad)).reshape(N_TILES, NB, KB)

    xs = x_src.reshape(2 * N_NODES, DH)

    out = _sc_segment(xs, i_src_p, i_dst_p, w_p)
    return _tc_post(x_dst, out, W_loop.T, W_intr.T)
